# baseline (speedup 1.0000x reference)
import functools

import jax
import jax.numpy as jnp
from jax import lax
from jax.experimental import pallas as pl
from jax.experimental.pallas import tpu as pltpu
from jax.experimental.pallas import tpu_sc as plsc


@functools.lru_cache(maxsize=None)
def _build(B, S, D):
    mesh = plsc.ScalarSubcoreMesh(axis_name="c", num_cores=1)

    @functools.partial(
        pl.kernel,
        mesh=mesh,
        out_type=jax.ShapeDtypeStruct((B, D), jnp.float32),
        scratch_types=[
            pltpu.SMEM((B,), jnp.int32),
            pltpu.SemaphoreType.DMA,
            pltpu.SemaphoreType.DMA,
        ],
    )
    def gather_kernel(flat_hbm, idx_hbm, out_hbm, ids_s, idx_sem, sem):
        pltpu.make_async_copy(idx_hbm, ids_s, idx_sem).start()
        pltpu.make_async_copy(idx_hbm, ids_s, idx_sem).wait()

        def body(b, _):
            gid = b * S + ids_s[b]
            pltpu.make_async_copy(
                flat_hbm.at[pl.ds(gid, 1)], out_hbm.at[pl.ds(b, 1)], sem
            ).start()
            return 0

        lax.fori_loop(0, B, body, 0)
        pltpu.make_async_copy(flat_hbm.at[pl.ds(0, B)], out_hbm, sem).wait()

    return gather_kernel


def kernel(hidden_states, input_id):
    B, S, D = hidden_states.shape
    flat = hidden_states.reshape(B * S, D)
    return _build(B, S, D)(flat, input_id.astype(jnp.int32))
